# initial kernel scaffold (unmeasured)
import jax
import jax.numpy as jnp
from jax import lax
from jax.experimental import pallas as pl
from jax.experimental.pallas import tpu as pltpu


def kernel(
    x,
):
    def body(*refs):
        pass

    out_shape = jax.ShapeDtypeStruct(..., jnp.float32)
    return pl.pallas_call(body, out_shape=out_shape)(...)


# baseline (device time: 435317 ns/iter reference)
import jax
import jax.numpy as jnp
from jax import lax
from jax.experimental import pallas as pl
from jax.experimental.pallas import tpu as pltpu

CHUNK = 1024


def kernel(x):
    m, n = x.shape
    nchunk = m // CHUNK

    def body(x_ref, out_ref, xbuf, cbuf, in_sem, out_sem, send_sems, recv_sems):
        my_x = lax.axis_index("x")
        my_y = lax.axis_index("y")
        my_z = lax.axis_index("z")
        partner = (my_x, 1 - my_y, my_z)

        barrier_sem = pltpu.get_barrier_semaphore()
        pl.semaphore_signal(
            barrier_sem, inc=1,
            device_id=partner, device_id_type=pl.DeviceIdType.MESH,
        )
        pl.semaphore_wait(barrier_sem, 1)

        for c in range(nchunk):
            slot = c % 2
            rows = pl.ds(c * CHUNK, CHUNK)

            cp_in = pltpu.make_async_copy(x_ref.at[rows], xbuf.at[slot], in_sem)
            cp_in.start()
            cp_in.wait()

            rdma = pltpu.make_async_remote_copy(
                src_ref=xbuf.at[slot],
                dst_ref=cbuf.at[slot],
                send_sem=send_sems.at[slot],
                recv_sem=recv_sems.at[slot],
                device_id=partner,
                device_id_type=pl.DeviceIdType.MESH,
            )
            rdma.start()
            rdma.wait()

            xbuf[slot] = xbuf[slot] + cbuf[slot]
            cp_out = pltpu.make_async_copy(xbuf.at[slot], out_ref.at[rows], out_sem)
            cp_out.start()
            cp_out.wait()

    return pl.pallas_call(
        body,
        out_shape=jax.ShapeDtypeStruct((m, n), x.dtype),
        in_specs=[pl.BlockSpec(memory_space=pl.ANY)],
        out_specs=pl.BlockSpec(memory_space=pl.ANY),
        scratch_shapes=[
            pltpu.VMEM((2, CHUNK, n), x.dtype),
            pltpu.VMEM((2, CHUNK, n), x.dtype),
            pltpu.SemaphoreType.DMA,
            pltpu.SemaphoreType.DMA,
            pltpu.SemaphoreType.DMA((2,)),
            pltpu.SemaphoreType.DMA((2,)),
        ],
        compiler_params=pltpu.CompilerParams(collective_id=0),
    )(x)


# device time: 224380 ns/iter; 1.9401x vs baseline; 1.9401x over previous
import jax
import jax.numpy as jnp
from jax import lax
from jax.experimental import pallas as pl
from jax.experimental.pallas import tpu as pltpu

CHUNK = 256
LS = 8


def kernel(x):
    m, n = x.shape
    half = m // 2
    nch = half // CHUNK
    ntot = 2 * nch

    def body(x_ref, out_ref, ybuf, xfbuf, lbuf,
             in_sems, out_sems, ysend, yrecv, xfsend, xfrecv):
        my_x = lax.axis_index("x")
        my_y = lax.axis_index("y")
        my_z = lax.axis_index("z")
        partner = (my_x, 1 - my_y, my_z)
        xnbr = (1 - my_x, my_y, my_z)
        y_base = my_x * half
        f_base = (1 - my_x) * half

        barrier_sem = pltpu.get_barrier_semaphore()
        for nbr in (partner, xnbr):
            pl.semaphore_signal(
                barrier_sem, inc=1,
                device_id=nbr, device_id_type=pl.DeviceIdType.MESH,
            )
        pl.semaphore_wait(barrier_sem, 2)

        y_rdmas = []
        for k in range(nch):
            r = pltpu.make_async_remote_copy(
                src_ref=x_ref.at[pl.ds(y_base + k * CHUNK, CHUNK)],
                dst_ref=ybuf.at[k],
                send_sem=ysend.at[k],
                recv_sem=yrecv.at[k],
                device_id=partner,
                device_id_type=pl.DeviceIdType.MESH,
            )
            r.start()
            y_rdmas.append(r)

        def add_rows(j):
            base = y_base if j < nch else f_base
            return base + (j % nch) * CHUNK

        cp_ins = {}

        def start_load(j):
            c = pltpu.make_async_copy(
                x_ref.at[pl.ds(add_rows(j), CHUNK)],
                lbuf.at[j % LS],
                in_sems.at[j % LS],
            )
            c.start()
            cp_ins[j] = c

        for j in range(min(LS, ntot)):
            start_load(j)

        xf_rdmas = []
        out_cps = {}
        for j in range(ntot):
            s = j % LS
            k = j % nch
            if j < nch:
                y_rdmas[k].wait_recv()
                rf = pltpu.make_async_remote_copy(
                    src_ref=ybuf.at[k],
                    dst_ref=xfbuf.at[k],
                    send_sem=xfsend.at[k],
                    recv_sem=xfrecv.at[k],
                    device_id=xnbr,
                    device_id_type=pl.DeviceIdType.MESH,
                )
                rf.start()
                xf_rdmas.append(rf)
                remote = ybuf[k]
            else:
                xf_rdmas[k].wait_recv()
                remote = xfbuf[k]

            cp_ins[j].wait()
            lbuf[s] = lbuf[s] + remote
            oc = pltpu.make_async_copy(
                lbuf.at[s],
                out_ref.at[pl.ds(add_rows(j), CHUNK)],
                out_sems.at[s],
            )
            oc.start()
            out_cps[j] = oc
            nj = j + LS
            if nj < ntot:
                out_cps[j].wait()
                start_load(nj)

        for j in range(max(0, ntot - LS), ntot):
            out_cps[j].wait()
        for r in y_rdmas:
            r.wait_send()
        for r in xf_rdmas:
            r.wait_send()

    return pl.pallas_call(
        body,
        out_shape=jax.ShapeDtypeStruct((m, n), x.dtype),
        in_specs=[pl.BlockSpec(memory_space=pl.ANY)],
        out_specs=pl.BlockSpec(memory_space=pl.ANY),
        scratch_shapes=[
            pltpu.VMEM((nch, CHUNK, n), x.dtype),
            pltpu.VMEM((nch, CHUNK, n), x.dtype),
            pltpu.VMEM((LS, CHUNK, n), x.dtype),
            pltpu.SemaphoreType.DMA((LS,)),
            pltpu.SemaphoreType.DMA((LS,)),
            pltpu.SemaphoreType.DMA((nch,)),
            pltpu.SemaphoreType.DMA((nch,)),
            pltpu.SemaphoreType.DMA((nch,)),
            pltpu.SemaphoreType.DMA((nch,)),
        ],
        compiler_params=pltpu.CompilerParams(
            collective_id=0, vmem_limit_bytes=60 * 1024 * 1024,
        ),
    )(x)


# device time: 176163 ns/iter; 2.4711x vs baseline; 1.2737x over previous
import jax
import jax.numpy as jnp
from jax import lax
from jax.experimental import pallas as pl
from jax.experimental.pallas import tpu as pltpu

CH = 128
LS = 8


def kernel(x):
    m, n = x.shape
    qrows = m // 4
    nch = qrows // CH
    hn = nch // 2
    nt = 4 * nch

    def body(x_ref, out_ref, ybuf, xbuf, zbuf, lbuf,
             in_sems, out_sems, ysend, yrecv, xsend, xrecv, zsend, zrecv):
        my_x = lax.axis_index("x")
        my_y = lax.axis_index("y")
        my_z = lax.axis_index("z")
        zl = lax.rem(my_z, 2)
        partner = (my_x, 1 - my_y, my_z)
        xn = (1 - my_x, my_y, my_z)
        zn = (my_x, my_y, my_z + 1 - 2 * zl)

        b_me = (2 * zl + my_x) * qrows
        b_xn = (2 * zl + (1 - my_x)) * qrows
        b_zn = (2 * (1 - zl) + my_x) * qrows
        b_dg = (2 * (1 - zl) + (1 - my_x)) * qrows

        barrier_sem = pltpu.get_barrier_semaphore()
        for nbr in (partner, xn, zn):
            pl.semaphore_signal(
                barrier_sem, inc=1,
                device_id=nbr, device_id_type=pl.DeviceIdType.MESH,
            )
        pl.semaphore_wait(barrier_sem, 3)

        y_rdmas = []
        for k in range(nch):
            r = pltpu.make_async_remote_copy(
                src_ref=x_ref.at[pl.ds(b_me + k * CH, CH)],
                dst_ref=ybuf.at[k],
                send_sem=ysend.at[k],
                recv_sem=yrecv.at[k],
                device_id=partner,
                device_id_type=pl.DeviceIdType.MESH,
            )
            r.start()
            y_rdmas.append(r)

        fwd_rdmas = []

        def fwd(src, dst, ssem, rsem, dev):
            r = pltpu.make_async_remote_copy(
                src_ref=src, dst_ref=dst, send_sem=ssem, recv_sem=rsem,
                device_id=dev, device_id_type=pl.DeviceIdType.MESH,
            )
            r.start()
            fwd_rdmas.append(r)

        def wait_recv(buf, rsem, dev):
            pltpu.make_async_remote_copy(
                src_ref=buf, dst_ref=buf, send_sem=rsem, recv_sem=rsem,
                device_id=dev, device_id_type=pl.DeviceIdType.MESH,
            ).wait_recv()

        rows_list = []
        pre_list = []
        rem_list = []

        for k in range(nch):
            rows_list.append(b_me + k * CH)
            rem_list.append(("y", k))

            def p(k=k):
                y_rdmas[k].wait_recv()
                fwd(ybuf.at[k], xbuf.at[k], xsend.at[k], xrecv.at[k], xn)
                fwd(ybuf.at[k], zbuf.at[k], zsend.at[k], zrecv.at[k], zn)
            pre_list.append(p)

        for j in range(hn):
            rows_list.append(b_zn + j * CH)
            rem_list.append(("z", j))

            def p(j=j):
                wait_recv(zbuf.at[j], zrecv.at[j], zn)
                fwd(zbuf.at[j], xbuf.at[nch + j],
                    xsend.at[nch + j], xrecv.at[nch + j], xn)
            pre_list.append(p)

        for j in range(hn, nch):
            rows_list.append(b_xn + j * CH)
            rem_list.append(("x", j))

            def p(j=j):
                wait_recv(xbuf.at[j], xrecv.at[j], xn)
                fwd(xbuf.at[j], zbuf.at[nch + j - hn],
                    zsend.at[nch + j - hn], zrecv.at[nch + j - hn], zn)
            pre_list.append(p)

        for j in range(hn):
            rows_list.append(b_xn + j * CH)
            rem_list.append(("x", j))

            def p(j=j):
                wait_recv(xbuf.at[j], xrecv.at[j], xn)
            pre_list.append(p)

        for j in range(hn, nch):
            rows_list.append(b_zn + j * CH)
            rem_list.append(("z", j))

            def p(j=j):
                wait_recv(zbuf.at[j], zrecv.at[j], zn)
            pre_list.append(p)

        for j in range(hn):
            rows_list.append(b_dg + j * CH)
            rem_list.append(("x", nch + j))

            def p(j=j):
                wait_recv(xbuf.at[nch + j], xrecv.at[nch + j], xn)
            pre_list.append(p)

            rows_list.append(b_dg + (hn + j) * CH)
            rem_list.append(("z", nch + j))

            def p(j=j):
                wait_recv(zbuf.at[nch + j], zrecv.at[nch + j], zn)
            pre_list.append(p)

        bufs = {"y": ybuf, "x": xbuf, "z": zbuf}

        cp_ins = {}

        def start_load(j):
            c = pltpu.make_async_copy(
                x_ref.at[pl.ds(rows_list[j], CH)],
                lbuf.at[j % LS],
                in_sems.at[j % LS],
            )
            c.start()
            cp_ins[j] = c

        for j in range(min(LS, nt)):
            start_load(j)

        out_cps = {}
        for j in range(nt):
            s = j % LS
            pre_list[j]()
            cp_ins[j].wait()
            which, idx = rem_list[j]
            lbuf[s] = lbuf[s] + bufs[which][idx]
            oc = pltpu.make_async_copy(
                lbuf.at[s],
                out_ref.at[pl.ds(rows_list[j], CH)],
                out_sems.at[s],
            )
            oc.start()
            out_cps[j] = oc
            if j + LS < nt:
                oc.wait()
                start_load(j + LS)

        for j in range(max(0, nt - LS), nt):
            out_cps[j].wait()
        for r in y_rdmas:
            r.wait_send()
        for r in fwd_rdmas:
            r.wait_send()

    nfl = nch + hn
    return pl.pallas_call(
        body,
        out_shape=jax.ShapeDtypeStruct((m, n), x.dtype),
        in_specs=[pl.BlockSpec(memory_space=pl.ANY)],
        out_specs=pl.BlockSpec(memory_space=pl.ANY),
        scratch_shapes=[
            pltpu.VMEM((nch, CH, n), x.dtype),
            pltpu.VMEM((nfl, CH, n), x.dtype),
            pltpu.VMEM((nfl, CH, n), x.dtype),
            pltpu.VMEM((LS, CH, n), x.dtype),
            pltpu.SemaphoreType.DMA((LS,)),
            pltpu.SemaphoreType.DMA((LS,)),
            pltpu.SemaphoreType.DMA((nch,)),
            pltpu.SemaphoreType.DMA((nch,)),
            pltpu.SemaphoreType.DMA((nfl,)),
            pltpu.SemaphoreType.DMA((nfl,)),
            pltpu.SemaphoreType.DMA((nfl,)),
            pltpu.SemaphoreType.DMA((nfl,)),
        ],
        compiler_params=pltpu.CompilerParams(
            collective_id=0, vmem_limit_bytes=60 * 1024 * 1024,
        ),
    )(x)


# device time: 166327 ns/iter; 2.6172x vs baseline; 1.0591x over previous
import jax
import jax.numpy as jnp
from jax import lax
from jax.experimental import pallas as pl
from jax.experimental.pallas import tpu as pltpu

CH = 128
LS = 4
S3 = (768, 640, 640)
OFF3 = (0, 768, 1408)

SZ = {rel: S3[rel % 3] for rel in range(12)}
NCK = {rel: SZ[rel] // CH for rel in range(12)}

Y_QUEUE = (1, 2, 0, 3)
YB_OFF = {1: 0, 2: 640, 0: 1280, 3: 2048}
YPOS = {1: 0, 2: 5, 0: 10, 3: 16}

R_QUEUE = (1, 10, 2, 0)
RPOS = {1: 0, 10: 5, 2: 10, 0: 15}
L_QUEUE = (1, 2, 5, 3)
LPOS = {1: 0, 2: 5, 5: 10, 3: 15}


ADD_ORDER = (
    (1, "y", 0), (10, "l", 0), (4, "r", 0),
    (2, "y", 640), (5, "r", 640),
    (0, "y", 1280), (7, "l", 640),
    (11, "l", 1280), (8, "r", 1280),
    (9, "l", 1920), (6, "r", 1920),
    (3, "y", 2048),
)


def kernel(x):
    m, n = x.shape

    def body(x_ref, out_ref, ybuf, linbuf, rinbuf, lbuf,
             in_sems, out_sems, ysend, yrecv, rsend, linrecv, lsend, rinrecv):
        my_x = lax.axis_index("x")
        my_y = lax.axis_index("y")
        my_z = lax.axis_index("z")
        zl = lax.rem(my_z, 2)
        zpz = my_z + 1 - 2 * zl
        xz = lax.rem(my_x + zl, 2)
        r_ring = 2 * zl + xz
        e = xz == 0

        partner = (my_x, 1 - my_y, my_z)
        xn = (1 - my_x, my_y, my_z)
        zn = (my_x, my_y, zpz)
        right_dev = (jnp.where(e, 1 - my_x, my_x), my_y,
                     jnp.where(e, my_z, zpz))
        left_dev = (jnp.where(e, my_x, 1 - my_x), my_y,
                    jnp.where(e, zpz, my_z))

        def off(rel):
            return 2048 * lax.rem(r_ring + rel // 3, 4) + OFF3[rel % 3]

        barrier_sem = pltpu.get_barrier_semaphore()
        for nbr in (partner, xn, zn):
            pl.semaphore_signal(
                barrier_sem, inc=1,
                device_id=nbr, device_id_type=pl.DeviceIdType.MESH,
            )
        pl.semaphore_wait(barrier_sem, 3)

        y_rd = {}
        y_rdmas = []
        p = 0
        for rel in Y_QUEUE:
            for c in range(NCK[rel]):
                rr = pltpu.make_async_remote_copy(
                    src_ref=x_ref.at[pl.ds(off(rel) + c * CH, CH)],
                    dst_ref=ybuf.at[pl.ds(YB_OFF[rel] + c * CH, CH)],
                    send_sem=ysend.at[p],
                    recv_sem=yrecv.at[p],
                    device_id=partner,
                    device_id_type=pl.DeviceIdType.MESH,
                )
                rr.start()
                y_rd[(rel, c)] = rr
                y_rdmas.append(rr)
                p += 1

        fwd_rdmas = []

        def rfwd(rel, c):
            if rel == 10:
                src = linbuf.at[pl.ds(0 + c * CH, CH)]
            else:
                src = ybuf.at[pl.ds(YB_OFF[rel] + c * CH, CH)]
            q = RPOS[rel] + c
            rr = pltpu.make_async_remote_copy(
                src_ref=src,
                dst_ref=linbuf.at[pl.ds(q * CH, CH)],
                send_sem=rsend.at[q],
                recv_sem=linrecv.at[q],
                device_id=right_dev,
                device_id_type=pl.DeviceIdType.MESH,
            )
            rr.start()
            fwd_rdmas.append(rr)

        def lfwd(rel, c):
            if rel == 5:
                src = rinbuf.at[pl.ds(640 + c * CH, CH)]
            else:
                src = ybuf.at[pl.ds(YB_OFF[rel] + c * CH, CH)]
            q = LPOS[rel] + c
            rr = pltpu.make_async_remote_copy(
                src_ref=src,
                dst_ref=rinbuf.at[pl.ds(q * CH, CH)],
                send_sem=lsend.at[q],
                recv_sem=rinrecv.at[q],
                device_id=left_dev,
                device_id_type=pl.DeviceIdType.MESH,
            )
            rr.start()
            fwd_rdmas.append(rr)

        def wait_lin(q):
            pltpu.make_async_remote_copy(
                src_ref=linbuf.at[pl.ds(q * CH, CH)],
                dst_ref=linbuf.at[pl.ds(q * CH, CH)],
                send_sem=linrecv.at[q], recv_sem=linrecv.at[q],
                device_id=left_dev, device_id_type=pl.DeviceIdType.MESH,
            ).wait_recv()

        def wait_rin(q):
            pltpu.make_async_remote_copy(
                src_ref=rinbuf.at[pl.ds(q * CH, CH)],
                dst_ref=rinbuf.at[pl.ds(q * CH, CH)],
                send_sem=rinrecv.at[q], recv_sem=rinrecv.at[q],
                device_id=right_dev, device_id_type=pl.DeviceIdType.MESH,
            ).wait_recv()

        bufs = {"y": ybuf, "l": linbuf, "r": rinbuf}
        cp_ins = {}
        out_cps = {}

        def start_load(aj):
            rel = ADD_ORDER[aj][0]
            c = pltpu.make_async_copy(
                x_ref.at[pl.ds(off(rel), SZ[rel])],
                lbuf.at[aj % LS, pl.ds(0, SZ[rel])],
                in_sems.at[aj % LS],
            )
            c.start()
            cp_ins[aj] = c

        def do_add(aj):
            rel, key, o = ADD_ORDER[aj]
            szr = SZ[rel]
            s = aj % LS
            cp_ins[aj].wait()
            lbuf[s, :szr] = lbuf[s, :szr] + bufs[key][o:o + szr]
            oc = pltpu.make_async_copy(
                lbuf.at[s, pl.ds(0, szr)],
                out_ref.at[pl.ds(off(rel), szr)],
                out_sems.at[s],
            )
            oc.start()
            out_cps[aj] = oc
            if aj + LS < 12:
                oc.wait()
                start_load(aj + LS)

        for aj in range(LS):
            start_load(aj)

        for c in range(5):
            y_rd[(1, c)].wait_recv()
            rfwd(1, c)
            lfwd(1, c)
        for c in range(5):
            wait_lin(c)
            rfwd(10, c)
        do_add(0)
        do_add(1)
        for c in range(5):
            wait_rin(c)
        do_add(2)
        for c in range(5):
            y_rd[(2, c)].wait_recv()
            rfwd(2, c)
            lfwd(2, c)
        for c in range(5):
            wait_rin(5 + c)
            lfwd(5, c)
        do_add(3)
        do_add(4)
        for c in range(6):
            y_rd[(0, c)].wait_recv()
            rfwd(0, c)
        do_add(5)
        for c in range(5):
            wait_lin(5 + c)
        do_add(6)
        for c in range(3):
            y_rd[(3, c)].wait_recv()
            lfwd(3, c)
        for c in range(5):
            wait_lin(10 + c)
        do_add(7)
        for c in range(5):
            wait_rin(10 + c)
        do_add(8)
        for c in range(3, 6):
            y_rd[(3, c)].wait_recv()
            lfwd(3, c)
        for c in range(6):
            wait_lin(15 + c)
        do_add(9)
        for c in range(6):
            wait_rin(15 + c)
        do_add(10)
        do_add(11)

        for aj in range(12 - LS, 12):
            out_cps[aj].wait()
        for rr in y_rdmas:
            rr.wait_send()
        for rr in fwd_rdmas:
            rr.wait_send()

    return pl.pallas_call(
        body,
        out_shape=jax.ShapeDtypeStruct((m, n), x.dtype),
        in_specs=[pl.BlockSpec(memory_space=pl.ANY)],
        out_specs=pl.BlockSpec(memory_space=pl.ANY),
        scratch_shapes=[
            pltpu.VMEM((2816, n), x.dtype),
            pltpu.VMEM((2688, n), x.dtype),
            pltpu.VMEM((2688, n), x.dtype),
            pltpu.VMEM((LS, 768, n), x.dtype),
            pltpu.SemaphoreType.DMA((LS,)),
            pltpu.SemaphoreType.DMA((LS,)),
            pltpu.SemaphoreType.DMA((22,)),
            pltpu.SemaphoreType.DMA((22,)),
            pltpu.SemaphoreType.DMA((21,)),
            pltpu.SemaphoreType.DMA((21,)),
            pltpu.SemaphoreType.DMA((21,)),
            pltpu.SemaphoreType.DMA((21,)),
        ],
        compiler_params=pltpu.CompilerParams(
            collective_id=0, vmem_limit_bytes=60 * 1024 * 1024,
        ),
    )(x)


# device time: 166277 ns/iter; 2.6180x vs baseline; 1.0003x over previous
import jax
import jax.numpy as jnp
from jax import lax
from jax.experimental import pallas as pl
from jax.experimental.pallas import tpu as pltpu

CH = 128
LS = 4
S3 = (768, 640, 640)
OFF3 = (0, 768, 1408)

SZ = {rel: S3[rel % 3] for rel in range(12)}
NCK = {rel: SZ[rel] // CH for rel in range(12)}

Y_QUEUE = (1, 2, 0, 3)
YB_OFF = {1: 0, 2: 640, 0: 1280, 3: 2048}
YPOS = {1: 0, 2: 5, 0: 10, 3: 16}

R_QUEUE = (1, 10, 2, 0)
RPOS = {1: 0, 10: 5, 2: 10, 0: 15}
L_QUEUE = (1, 2, 5, 3)
LPOS = {1: 0, 2: 5, 5: 10, 3: 15}


ADD_ORDER = (
    (1, "y", 0), (10, "l", 0), (4, "r", 0),
    (2, "y", 640), (5, "r", 640),
    (0, "y", 1280), (7, "l", 640),
    (11, "l", 1280), (8, "r", 1280),
    (9, "l", 1920), (6, "r", 1920),
    (3, "y", 2048),
)


def kernel(x):
    m, n = x.shape

    def body(x_ref, out_ref, ybuf, linbuf, rinbuf, lbuf,
             in_sems, out_sems, ysend, yrecv, rsend, linrecv, lsend, rinrecv):
        my_x = lax.axis_index("x")
        my_y = lax.axis_index("y")
        my_z = lax.axis_index("z")
        zl = lax.rem(my_z, 2)
        zpz = my_z + 1 - 2 * zl
        xz = lax.rem(my_x + zl, 2)
        r_ring = 2 * zl + xz
        e = xz == 0

        partner = (my_x, 1 - my_y, my_z)
        xn = (1 - my_x, my_y, my_z)
        zn = (my_x, my_y, zpz)
        right_dev = (jnp.where(e, 1 - my_x, my_x), my_y,
                     jnp.where(e, my_z, zpz))
        left_dev = (jnp.where(e, my_x, 1 - my_x), my_y,
                    jnp.where(e, zpz, my_z))

        def off(rel):
            return 2048 * lax.rem(r_ring + rel // 3, 4) + OFF3[rel % 3]

        barrier_sem = pltpu.get_barrier_semaphore()
        for nbr in (partner, xn, zn):
            pl.semaphore_signal(
                barrier_sem, inc=1,
                device_id=nbr, device_id_type=pl.DeviceIdType.MESH,
            )
        pl.semaphore_wait(barrier_sem, 3)

        y_rd = {}
        y_rdmas = []
        p = 0
        for rel in Y_QUEUE:
            for c in range(NCK[rel]):
                rr = pltpu.make_async_remote_copy(
                    src_ref=x_ref.at[pl.ds(off(rel) + c * CH, CH)],
                    dst_ref=ybuf.at[pl.ds(YB_OFF[rel] + c * CH, CH)],
                    send_sem=ysend.at[p],
                    recv_sem=yrecv.at[p],
                    device_id=partner,
                    device_id_type=pl.DeviceIdType.MESH,
                )
                rr.start()
                y_rd[(rel, c)] = rr
                y_rdmas.append(rr)
                p += 1

        fwd_rdmas = []

        def rfwd(rel, c):
            if rel == 10:
                src = linbuf.at[pl.ds(0 + c * CH, CH)]
            else:
                src = ybuf.at[pl.ds(YB_OFF[rel] + c * CH, CH)]
            q = RPOS[rel] + c
            rr = pltpu.make_async_remote_copy(
                src_ref=src,
                dst_ref=linbuf.at[pl.ds(q * CH, CH)],
                send_sem=rsend.at[q],
                recv_sem=linrecv.at[q],
                device_id=right_dev,
                device_id_type=pl.DeviceIdType.MESH,
            )
            rr.start()
            fwd_rdmas.append(rr)

        def lfwd(rel, c):
            if rel == 5:
                src = rinbuf.at[pl.ds(640 + c * CH, CH)]
            else:
                src = ybuf.at[pl.ds(YB_OFF[rel] + c * CH, CH)]
            q = LPOS[rel] + c
            rr = pltpu.make_async_remote_copy(
                src_ref=src,
                dst_ref=rinbuf.at[pl.ds(q * CH, CH)],
                send_sem=lsend.at[q],
                recv_sem=rinrecv.at[q],
                device_id=left_dev,
                device_id_type=pl.DeviceIdType.MESH,
            )
            rr.start()
            fwd_rdmas.append(rr)

        def wait_lin(q):
            pltpu.make_async_remote_copy(
                src_ref=linbuf.at[pl.ds(q * CH, CH)],
                dst_ref=linbuf.at[pl.ds(q * CH, CH)],
                send_sem=linrecv.at[q], recv_sem=linrecv.at[q],
                device_id=left_dev, device_id_type=pl.DeviceIdType.MESH,
            ).wait_recv()

        def wait_rin(q):
            pltpu.make_async_remote_copy(
                src_ref=rinbuf.at[pl.ds(q * CH, CH)],
                dst_ref=rinbuf.at[pl.ds(q * CH, CH)],
                send_sem=rinrecv.at[q], recv_sem=rinrecv.at[q],
                device_id=right_dev, device_id_type=pl.DeviceIdType.MESH,
            ).wait_recv()

        bufs = {"y": ybuf, "l": linbuf, "r": rinbuf}
        cp_ins = {}
        out_cps = {}

        def start_load(aj):
            rel = ADD_ORDER[aj][0]
            c = pltpu.make_async_copy(
                x_ref.at[pl.ds(off(rel), SZ[rel])],
                lbuf.at[aj % LS, pl.ds(0, SZ[rel])],
                in_sems.at[aj % LS],
            )
            c.start()
            cp_ins[aj] = c

        def do_add(aj):
            rel, key, o = ADD_ORDER[aj]
            szr = SZ[rel]
            s = aj % LS
            cp_ins[aj].wait()
            lbuf[s, :szr] = lbuf[s, :szr] + bufs[key][o:o + szr]
            oc = pltpu.make_async_copy(
                lbuf.at[s, pl.ds(0, szr)],
                out_ref.at[pl.ds(off(rel), szr)],
                out_sems.at[s],
            )
            oc.start()
            out_cps[aj] = oc
            if aj + LS < 12:
                oc.wait()
                start_load(aj + LS)

        for aj in range(LS):
            start_load(aj)

        for c in range(5):
            y_rd[(1, c)].wait_recv()
            rfwd(1, c)
            lfwd(1, c)
        for c in range(5):
            wait_lin(c)
            rfwd(10, c)
        do_add(0)
        do_add(1)
        for c in range(5):
            wait_rin(c)
        do_add(2)
        for c in range(5):
            y_rd[(2, c)].wait_recv()
            rfwd(2, c)
            lfwd(2, c)
        for c in range(5):
            wait_rin(5 + c)
            lfwd(5, c)
        do_add(3)
        do_add(4)
        for c in range(6):
            y_rd[(0, c)].wait_recv()
            rfwd(0, c)
        do_add(5)
        for c in range(5):
            wait_lin(5 + c)
        do_add(6)
        for c in range(3):
            y_rd[(3, c)].wait_recv()
            lfwd(3, c)
        for c in range(5):
            wait_lin(10 + c)
        do_add(7)
        for c in range(5):
            wait_rin(10 + c)
        do_add(8)
        for c in range(3, 6):
            y_rd[(3, c)].wait_recv()
            lfwd(3, c)
        for c in range(6):
            wait_lin(15 + c)
        do_add(9)
        for c in range(6):
            wait_rin(15 + c)
        do_add(10)
        do_add(11)

        for aj in range(12 - LS, 12):
            out_cps[aj].wait()
        for rr in y_rdmas:
            rr.wait_send()
        for rr in fwd_rdmas:
            rr.wait_send()

    return pl.pallas_call(
        body,
        out_shape=jax.ShapeDtypeStruct((m, n), x.dtype),
        in_specs=[pl.BlockSpec(memory_space=pltpu.MemorySpace.HBM)],
        out_specs=pl.BlockSpec(memory_space=pltpu.MemorySpace.HBM),
        scratch_shapes=[
            pltpu.VMEM((2816, n), x.dtype),
            pltpu.VMEM((2688, n), x.dtype),
            pltpu.VMEM((2688, n), x.dtype),
            pltpu.VMEM((LS, 768, n), x.dtype),
            pltpu.SemaphoreType.DMA((LS,)),
            pltpu.SemaphoreType.DMA((LS,)),
            pltpu.SemaphoreType.DMA((22,)),
            pltpu.SemaphoreType.DMA((22,)),
            pltpu.SemaphoreType.DMA((21,)),
            pltpu.SemaphoreType.DMA((21,)),
            pltpu.SemaphoreType.DMA((21,)),
            pltpu.SemaphoreType.DMA((21,)),
        ],
        compiler_params=pltpu.CompilerParams(
            collective_id=0, vmem_limit_bytes=60 * 1024 * 1024,
        ),
    )(x)
